# baseline (speedup 1.0000x reference)
import functools

import numpy as np
import jax
import jax.numpy as jnp
from jax import lax
from jax.experimental import pallas as pl
from jax.experimental.pallas import tpu as pltpu
from jax.experimental.pallas import tpu_sc as plsc

_NUM_SELECT = 50000
_NUM_COLS = 100000
_NUM_ROWS = 128
_CHUNK = 2000
_NUM_CHUNKS = _NUM_SELECT // _CHUNK
_LANES = 16


def _rotl(x, r):
    return ((x << np.uint32(r)) | (x >> np.uint32(32 - r))).astype(np.uint32)


def _threefry2x32(k0, k1, x0, x1):
    rot_a = (13, 15, 26, 6)
    rot_b = (17, 29, 16, 24)
    ks = [np.uint32(k0), np.uint32(k1),
          np.uint32(np.uint32(k0) ^ np.uint32(k1) ^ np.uint32(0x1BD11BDA))]
    x0 = (x0 + ks[0]).astype(np.uint32)
    x1 = (x1 + ks[1]).astype(np.uint32)
    for i, rots in enumerate((rot_a, rot_b, rot_a, rot_b, rot_a)):
        for r in rots:
            x0 = (x0 + x1).astype(np.uint32)
            x1 = _rotl(x1, r)
            x1 = x1 ^ x0
        x0 = (x0 + ks[(i + 1) % 3]).astype(np.uint32)
        x1 = (x1 + ks[(i + 2) % 3] + np.uint32(i + 1)).astype(np.uint32)
    return x0, x1


def _random_bits(k0, k1, n):
    hi = np.zeros(n, np.uint32)
    lo = np.arange(n, dtype=np.uint32)
    b0, b1 = _threefry2x32(k0, k1, hi, lo)
    return b0 ^ b1


def _split(k0, k1):
    hi = np.zeros(2, np.uint32)
    lo = np.arange(2, dtype=np.uint32)
    b0, b1 = _threefry2x32(k0, k1, hi, lo)
    return (b0[0], b1[0]), (b0[1], b1[1])


@functools.cache
def _select_indices() -> np.ndarray:
    n = _NUM_COLS
    key = (np.uint32(0), np.uint32(42))
    num_rounds = int(np.ceil(3 * np.log(max(1, n)) /
                             np.log(np.iinfo(np.uint32).max)))
    x = np.arange(n, dtype=np.int32)
    for _ in range(num_rounds):
        key, subkey = _split(*key)
        sort_keys = _random_bits(subkey[0], subkey[1], n)
        x = x[np.argsort(sort_keys, kind="stable")]
    return np.ascontiguousarray(x[:_NUM_SELECT])


def _sc_gather(x, idx):
    info = plsc.get_sparse_core_info()
    ncores, nsub = info.num_cores, info.num_subcores
    nworkers = ncores * nsub
    rows_per_w = _NUM_ROWS // nworkers
    mesh = plsc.VectorSubcoreMesh(core_axis_name="c", subcore_axis_name="s")

    @functools.partial(
        pl.kernel,
        mesh=mesh,
        compiler_params=pltpu.CompilerParams(
            use_tc_tiling_on_sc=False, needs_layout_passes=False
        ),
        out_type=jax.ShapeDtypeStruct((_NUM_ROWS, _NUM_SELECT), jnp.float32),
        scratch_types=[
            pltpu.VMEM((_NUM_COLS,), jnp.float32),
            pltpu.VMEM((2, _CHUNK), jnp.int32),
            pltpu.VMEM((2, _CHUNK), jnp.float32),
            pltpu.SemaphoreType.DMA,
            pltpu.SemaphoreType.DMA,
            pltpu.SemaphoreType.DMA,
            pltpu.SemaphoreType.DMA,
            pltpu.SemaphoreType.DMA,
        ],
    )
    def k(x_hbm, idx_hbm, out_hbm, row_v, idx_v, val_v,
          sem_row, sem_i0, sem_i1, sem_o0, sem_o1):
        wid = lax.axis_index("s") * ncores + lax.axis_index("c")
        idx_sems = (sem_i0, sem_i1)
        out_sems = (sem_o0, sem_o1)

        def gather_chunk(b):
            def body(j, carry):
                iv = idx_v[b, pl.ds(j * _LANES, _LANES)]
                val_v[b, pl.ds(j * _LANES, _LANES)] = plsc.load_gather(
                    row_v, [iv]
                )
                return carry

            lax.fori_loop(0, _CHUNK // _LANES, body, 0, unroll=8)

        pending_out = []
        row_copy = pltpu.async_copy(
            x_hbm.at[wid * rows_per_w], row_v, sem_row
        )
        idx_copy = pltpu.async_copy(
            idx_hbm.at[pl.ds(0, _CHUNK)], idx_v.at[0], idx_sems[0]
        )
        row_copy.wait()
        total = rows_per_w * _NUM_CHUNKS
        for gi in range(total):
            t, c = divmod(gi, _NUM_CHUNKS)
            r = wid * rows_per_w + t
            b = gi % 2
            idx_copy.wait()
            if gi + 1 < total:
                nc = (gi + 1) % _NUM_CHUNKS
                idx_copy = pltpu.async_copy(
                    idx_hbm.at[pl.ds(nc * _CHUNK, _CHUNK)],
                    idx_v.at[1 - b],
                    idx_sems[1 - b],
                )
            if len(pending_out) >= 2:
                pending_out.pop(0).wait()
            gather_chunk(b)
            pending_out.append(
                pltpu.async_copy(
                    val_v.at[b],
                    out_hbm.at[r, pl.ds(c * _CHUNK, _CHUNK)],
                    out_sems[b],
                )
            )
            if c == _NUM_CHUNKS - 1 and t + 1 < rows_per_w:
                row_copy = pltpu.async_copy(
                    x_hbm.at[r + 1], row_v, sem_row
                )
                row_copy.wait()
        for h in pending_out:
            h.wait()

    return k(x, idx)


def kernel(x):
    idx = jnp.asarray(_select_indices())
    return _sc_gather(x, idx)


# speedup vs baseline: 1.1141x; 1.1141x over previous
import functools

import numpy as np
import jax
import jax.numpy as jnp
from jax import lax
from jax.experimental import pallas as pl
from jax.experimental.pallas import tpu as pltpu
from jax.experimental.pallas import tpu_sc as plsc

_NUM_SELECT = 50000
_NUM_COLS = 100000
_NUM_ROWS = 128
_CHUNK = 2000
_NUM_CHUNKS = _NUM_SELECT // _CHUNK
_LANES = 16


def _rotl(x, r):
    return ((x << np.uint32(r)) | (x >> np.uint32(32 - r))).astype(np.uint32)


def _threefry2x32(k0, k1, x0, x1):
    rot_a = (13, 15, 26, 6)
    rot_b = (17, 29, 16, 24)
    ks = [np.uint32(k0), np.uint32(k1),
          np.uint32(np.uint32(k0) ^ np.uint32(k1) ^ np.uint32(0x1BD11BDA))]
    x0 = (x0 + ks[0]).astype(np.uint32)
    x1 = (x1 + ks[1]).astype(np.uint32)
    for i, rots in enumerate((rot_a, rot_b, rot_a, rot_b, rot_a)):
        for r in rots:
            x0 = (x0 + x1).astype(np.uint32)
            x1 = _rotl(x1, r)
            x1 = x1 ^ x0
        x0 = (x0 + ks[(i + 1) % 3]).astype(np.uint32)
        x1 = (x1 + ks[(i + 2) % 3] + np.uint32(i + 1)).astype(np.uint32)
    return x0, x1


def _random_bits(k0, k1, n):
    hi = np.zeros(n, np.uint32)
    lo = np.arange(n, dtype=np.uint32)
    b0, b1 = _threefry2x32(k0, k1, hi, lo)
    return b0 ^ b1


def _split(k0, k1):
    hi = np.zeros(2, np.uint32)
    lo = np.arange(2, dtype=np.uint32)
    b0, b1 = _threefry2x32(k0, k1, hi, lo)
    return (b0[0], b1[0]), (b0[1], b1[1])


@functools.cache
def _select_indices() -> np.ndarray:
    n = _NUM_COLS
    key = (np.uint32(0), np.uint32(42))
    num_rounds = int(np.ceil(3 * np.log(max(1, n)) /
                             np.log(np.iinfo(np.uint32).max)))
    x = np.arange(n, dtype=np.int32)
    for _ in range(num_rounds):
        key, subkey = _split(*key)
        sort_keys = _random_bits(subkey[0], subkey[1], n)
        x = x[np.argsort(sort_keys, kind="stable")]
    return np.ascontiguousarray(x[:_NUM_SELECT])


def _sc_gather(x, idx):
    info = plsc.get_sparse_core_info()
    ncores, nsub = info.num_cores, info.num_subcores
    nworkers = ncores * nsub
    rows_per_w = _NUM_ROWS // nworkers
    mesh = plsc.VectorSubcoreMesh(core_axis_name="c", subcore_axis_name="s")

    @functools.partial(
        pl.kernel,
        mesh=mesh,
        compiler_params=pltpu.CompilerParams(
            use_tc_tiling_on_sc=False, needs_layout_passes=False
        ),
        out_type=jax.ShapeDtypeStruct((_NUM_ROWS, _NUM_SELECT), jnp.float32),
        scratch_types=[
            pltpu.VMEM((_NUM_COLS,), jnp.float32),
            pltpu.VMEM((2, _CHUNK), jnp.int32),
            pltpu.VMEM((2, _CHUNK), jnp.float32),
            pltpu.SemaphoreType.DMA,
            pltpu.SemaphoreType.DMA,
            pltpu.SemaphoreType.DMA,
            pltpu.SemaphoreType.DMA,
            pltpu.SemaphoreType.DMA,
        ],
    )
    def k(x_hbm, idx_hbm, out_hbm, row_v, idx_v, val_v,
          sem_row, sem_i0, sem_i1, sem_o0, sem_o1):
        wid = lax.axis_index("s") * ncores + lax.axis_index("c")
        idx_sems = (sem_i0, sem_i1)
        out_sems = (sem_o0, sem_o1)

        def gather_chunk(b):
            @plsc.parallel_loop(0, _CHUNK // _LANES, unroll=4)
            def body(j):
                iv = idx_v[b, pl.ds(j * _LANES, _LANES)]
                val_v[b, pl.ds(j * _LANES, _LANES)] = plsc.load_gather(
                    row_v, [iv]
                )

        pending_out = []
        row_copy = pltpu.async_copy(
            x_hbm.at[wid * rows_per_w], row_v, sem_row
        )
        idx_copy = pltpu.async_copy(
            idx_hbm.at[pl.ds(0, _CHUNK)], idx_v.at[0], idx_sems[0]
        )
        row_copy.wait()
        total = rows_per_w * _NUM_CHUNKS
        for gi in range(total):
            t, c = divmod(gi, _NUM_CHUNKS)
            r = wid * rows_per_w + t
            b = gi % 2
            idx_copy.wait()
            if gi + 1 < total:
                nc = (gi + 1) % _NUM_CHUNKS
                idx_copy = pltpu.async_copy(
                    idx_hbm.at[pl.ds(nc * _CHUNK, _CHUNK)],
                    idx_v.at[1 - b],
                    idx_sems[1 - b],
                )
            if len(pending_out) >= 2:
                pending_out.pop(0).wait()
            gather_chunk(b)
            pending_out.append(
                pltpu.async_copy(
                    val_v.at[b],
                    out_hbm.at[r, pl.ds(c * _CHUNK, _CHUNK)],
                    out_sems[b],
                )
            )
            if c == _NUM_CHUNKS - 1 and t + 1 < rows_per_w:
                row_copy = pltpu.async_copy(
                    x_hbm.at[r + 1], row_v, sem_row
                )
                row_copy.wait()
        for h in pending_out:
            h.wait()

    return k(x, idx)


def kernel(x):
    idx = jnp.asarray(_select_indices())
    return _sc_gather(x, idx)


# speedup vs baseline: 7.4801x; 6.7141x over previous
import functools

import numpy as np
import jax
import jax.numpy as jnp
from jax import lax
from jax.experimental import pallas as pl
from jax.experimental.pallas import tpu as pltpu
from jax.experimental.pallas import tpu_sc as plsc

_NUM_SELECT = 50000
_NUM_COLS = 100000
_NUM_ROWS = 128
_CHUNK = 2000
_NUM_CHUNKS = _NUM_SELECT // _CHUNK
_LANES = 16


def _rotl(x, r):
    return ((x << np.uint32(r)) | (x >> np.uint32(32 - r))).astype(np.uint32)


def _threefry2x32(k0, k1, x0, x1):
    rot_a = (13, 15, 26, 6)
    rot_b = (17, 29, 16, 24)
    ks = [np.uint32(k0), np.uint32(k1),
          np.uint32(np.uint32(k0) ^ np.uint32(k1) ^ np.uint32(0x1BD11BDA))]
    x0 = (x0 + ks[0]).astype(np.uint32)
    x1 = (x1 + ks[1]).astype(np.uint32)
    for i, rots in enumerate((rot_a, rot_b, rot_a, rot_b, rot_a)):
        for r in rots:
            x0 = (x0 + x1).astype(np.uint32)
            x1 = _rotl(x1, r)
            x1 = x1 ^ x0
        x0 = (x0 + ks[(i + 1) % 3]).astype(np.uint32)
        x1 = (x1 + ks[(i + 2) % 3] + np.uint32(i + 1)).astype(np.uint32)
    return x0, x1


def _random_bits(k0, k1, n):
    hi = np.zeros(n, np.uint32)
    lo = np.arange(n, dtype=np.uint32)
    b0, b1 = _threefry2x32(k0, k1, hi, lo)
    return b0 ^ b1


def _split(k0, k1):
    hi = np.zeros(2, np.uint32)
    lo = np.arange(2, dtype=np.uint32)
    b0, b1 = _threefry2x32(k0, k1, hi, lo)
    return (b0[0], b1[0]), (b0[1], b1[1])


@functools.cache
def _select_indices() -> np.ndarray:
    n = _NUM_COLS
    key = (np.uint32(0), np.uint32(42))
    num_rounds = int(np.ceil(3 * np.log(max(1, n)) /
                             np.log(np.iinfo(np.uint32).max)))
    x = np.arange(n, dtype=np.int32)
    for _ in range(num_rounds):
        key, subkey = _split(*key)
        sort_keys = _random_bits(subkey[0], subkey[1], n)
        x = x[np.argsort(sort_keys, kind="stable")]
    return np.ascontiguousarray(x[:_NUM_SELECT])


_NBATCH = 391
_MAXK = 13


@functools.cache
def _batched_indices() -> np.ndarray:
    idx = np.zeros((_NBATCH * 128,), dtype=np.int32)
    idx[:_NUM_SELECT] = _select_indices()
    arr = np.zeros((32, _MAXK, 128), dtype=np.int32)
    for w in range(32):
        for k in range(_MAXK):
            bid = w + 32 * k
            if bid < _NBATCH:
                arr[w, k] = idx[bid * 128:(bid + 1) * 128]
    return arr


def _sc_gather_t(xt, idxb):
    mesh = plsc.VectorSubcoreMesh(core_axis_name="c", subcore_axis_name="s")

    @functools.partial(
        pl.kernel,
        mesh=mesh,
        out_type=jax.ShapeDtypeStruct((_NUM_SELECT, 128), jnp.float32),
        scratch_types=[
            pltpu.VMEM((_MAXK, 128), jnp.int32),
            pltpu.VMEM((2, 128, 128), jnp.float32),
            pltpu.SemaphoreType.DMA,
            pltpu.SemaphoreType.DMA,
            pltpu.SemaphoreType.DMA,
            pltpu.SemaphoreType.DMA,
            pltpu.SemaphoreType.DMA,
        ],
    )
    def k(xt_hbm, idxb_hbm, out_hbm, idxv, dstv,
          sem_i, sem_g0, sem_g1, sem_o0, sem_o1):
        ci = lax.axis_index("c")
        si = lax.axis_index("s")
        w = si * 2 + ci
        g_sems = (sem_g0, sem_g1)
        o_sems = (sem_o0, sem_o1)

        pltpu.async_copy(idxb_hbm.at[w], idxv, sem_i).wait()

        pending = [None, None]
        gathers = []
        for kk in range(12):
            b = kk % 2
            if pending[b] is not None:
                pending[b].wait()
                pending[b] = None
            gathers.append(
                pltpu.async_copy(
                    xt_hbm.at[idxv.at[kk]], dstv.at[b], g_sems[b]
                )
            )
            if kk >= 1:
                gathers[kk - 1].wait()
                roff = pl.multiple_of((w + 32 * (kk - 1)) * 128, 128)
                pending[1 - b] = pltpu.async_copy(
                    dstv.at[1 - b],
                    out_hbm.at[pl.ds(roff, 128)],
                    o_sems[1 - b],
                )
        gathers[11].wait()
        roff = pl.multiple_of((w + 32 * 11) * 128, 128)
        pending[1] = pltpu.async_copy(
            dstv.at[1], out_hbm.at[pl.ds(roff, 128)], o_sems[1]
        )
        for h in pending:
            if h is not None:
                h.wait()

        @pl.when(w < 6)
        def _():
            cp = pltpu.async_copy(
                xt_hbm.at[idxv.at[12]], dstv.at[0], g_sems[0]
            )
            cp.wait()
            roff = pl.multiple_of((w + 32 * 12) * 128, 128)
            pltpu.async_copy(
                dstv.at[0], out_hbm.at[pl.ds(roff, 128)], o_sems[0]
            ).wait()

        @pl.when(w == 6)
        def _():
            cp = pltpu.async_copy(
                xt_hbm.at[idxv.at[12]], dstv.at[0], g_sems[0]
            )
            cp.wait()
            pltpu.async_copy(
                dstv.at[0, pl.ds(0, 80)],
                out_hbm.at[pl.ds(49920, 80)],
                o_sems[0],
            ).wait()

    return k(xt, idxb)


def kernel(x):
    idxb = jnp.asarray(_batched_indices())
    out_t = _sc_gather_t(x.T, idxb)
    return out_t.T
